# baseline (speedup 1.0000x reference)
import functools

import jax
import jax.numpy as jnp
from jax import lax
from jax.experimental import pallas as pl
from jax.experimental.pallas import tpu as pltpu
from jax.experimental.pallas import tpu_sc as plsc

H = 128
NUM_LAYERS = 3
NUM_CLASSES = 97
NC = 2
NS = 16
NW = NC * NS

NODE_N = {"occ": 50000, "chord": 97, "sec": 2500, "note": 128, "song": 1000}
NODE_FEAT = {"occ": 32, "chord": 16, "sec": 8, "note": 12, "song": 8}
NODE_PAD = {"occ": 50176, "chord": 128, "sec": 2560, "note": 256, "song": 1024}

OCC_CHUNKS = 5
OCC_ROWS = 10240
OCC_CPAD = OCC_ROWS + 128


def _occ_chunk_rows(ci):
    lo = ci * OCC_ROWS
    return min(OCC_ROWS, NODE_PAD["occ"] - lo)

REL_LIST = [
    ("next", "occ", "occ"),
    ("instance_of", "occ", "chord"),
    ("inst_rev", "chord", "occ"),
    ("in_section", "occ", "sec"),
    ("sec_rev", "sec", "occ"),
    ("next_section", "sec", "sec"),
    ("chord_contains", "chord", "note"),
    ("note_in_chord", "note", "chord"),
    ("belongs_to", "occ", "song"),
    ("song_rev", "song", "occ"),
]
REL_E = {"next": 50000, "instance_of": 50000, "inst_rev": 50000,
         "in_section": 50000, "sec_rev": 50000, "next_section": 2500,
         "chord_contains": 512, "note_in_chord": 512, "belongs_to": 50000,
         "song_rev": 50000}

DST_RELS = {
    "occ": ["next", "inst_rev", "sec_rev", "song_rev"],
    "chord": ["instance_of", "note_in_chord"],
    "sec": ["in_section", "next_section"],
    "note": ["chord_contains"],
    "song": ["belongs_to"],
}


def _edge_plan(e):
    if e >= 16384:
        b = 256
    elif e >= 2048:
        b = 128
    else:
        b = 16
    per_worker = -(-e // NW)
    nblk = -(-per_worker // b)
    return b, nblk, NW * nblk * b


def _sc_segsum(table, idx, dst, zeros, n_dst_pad, b, nblk):
    rows_w = n_dst_pad // NS
    mesh = plsc.VectorSubcoreMesh(core_axis_name="c", subcore_axis_name="s",
                                  num_cores=NC)

    @functools.partial(
        pl.kernel, mesh=mesh,
        out_type=jax.ShapeDtypeStruct((NC, n_dst_pad, H), jnp.float32),
        scratch_types=[
            pltpu.VMEM((b,), jnp.int32),
            pltpu.VMEM((b,), jnp.int32),
            pltpu.VMEM((b, H), jnp.float32),
            pltpu.VMEM_SHARED((n_dst_pad, H), jnp.float32),
            pltpu.SemaphoreType.DMA,
        ])
    def k(table_h, idx_h, dst_h, zeros_h, out_h, idxv, dstv, rowsv, acc, sem):
        cid = lax.axis_index("c")
        sid = lax.axis_index("s")
        wid = cid * NS + sid
        pltpu.sync_copy(zeros_h.at[pl.ds(sid * rows_w, rows_w)],
                        acc.at[pl.ds(sid * rows_w, rows_w)])
        plsc.subcore_barrier()
        for kb in range(nblk):
            base = (wid * nblk + kb) * b
            pltpu.sync_copy(idx_h.at[pl.ds(base, b)], idxv)
            pltpu.sync_copy(dst_h.at[pl.ds(base, b)], dstv)
            pltpu.async_copy(table_h.at[idxv], rowsv, sem).wait()
            pltpu.sync_copy(rowsv, acc.at[dstv], add=True)
        plsc.subcore_barrier()
        pltpu.sync_copy(acc.at[pl.ds(sid * rows_w, rows_w)],
                        out_h.at[cid, pl.ds(sid * rows_w, rows_w)])

    return k(table, idx, dst, zeros)


def _tc_linear(x, w, bias, blk):
    n = x.shape[0]
    kdim, m = w.shape

    def body(x_ref, w_ref, b_ref, o_ref):
        o_ref[...] = (jnp.dot(x_ref[...], w_ref[...],
                              preferred_element_type=jnp.float32)
                      + b_ref[...])

    return pl.pallas_call(
        body,
        grid=(n // blk,),
        in_specs=[pl.BlockSpec((blk, kdim), lambda i: (i, 0)),
                  pl.BlockSpec((kdim, m), lambda i: (0, 0)),
                  pl.BlockSpec((1, m), lambda i: (0, 0))],
        out_specs=pl.BlockSpec((blk, m), lambda i: (i, 0)),
        out_shape=jax.ShapeDtypeStruct((n, m), jnp.float32),
    )(x, w, bias)


def _tc_combine(aggs, cnts, h, wl_stack, wr_sum, bl_sum, g, bb, n_out, blk):
    nrel = len(aggs)

    def body(*refs):
        agg_refs = refs[:nrel]
        cnt_refs = refs[nrel:2 * nrel]
        h_ref, wl_ref, wr_ref, bs_ref, g_ref, bb_ref, o_ref = refs[2 * nrel:]
        hblk = h_ref[...]
        wl = wl_ref[...]
        acc = jnp.dot(hblk, wr_ref[...], preferred_element_type=jnp.float32)
        acc = acc + bs_ref[...]
        for r in range(nrel):
            cv = cnt_refs[r][...]
            cv = cv[0] + cv[1]
            inv = 1.0 / jnp.maximum(cv[:, 0:1], 1.0)
            av = agg_refs[r][...]
            av = (av[0] + av[1]) * inv
            acc = acc + jnp.dot(av, wl[r],
                                preferred_element_type=jnp.float32)
        acc = acc * (1.0 / nrel)
        m = jnp.mean(acc, axis=-1, keepdims=True)
        v = jnp.mean((acc - m) * (acc - m), axis=-1, keepdims=True)
        y = (acc - m) * lax.rsqrt(v + 1e-5) * g_ref[...] + bb_ref[...]
        o_ref[...] = y + hblk

    in_specs = (
        [pl.BlockSpec((NC, blk, H), lambda i: (0, i, 0)) for _ in aggs]
        + [pl.BlockSpec((NC, blk, H), lambda i: (0, i, 0)) for _ in cnts]
        + [pl.BlockSpec((blk, H), lambda i: (i, 0)),
           pl.BlockSpec((nrel, H, H), lambda i: (0, 0, 0)),
           pl.BlockSpec((H, H), lambda i: (0, 0)),
           pl.BlockSpec((1, H), lambda i: (0, 0)),
           pl.BlockSpec((1, H), lambda i: (0, 0)),
           pl.BlockSpec((1, H), lambda i: (0, 0))]
    )
    return pl.pallas_call(
        body,
        grid=(n_out // blk,),
        in_specs=in_specs,
        out_specs=pl.BlockSpec((blk, H), lambda i: (i, 0)),
        out_shape=jax.ShapeDtypeStruct((n_out, H), jnp.float32),
    )(*aggs, *cnts, h, wl_stack, wr_sum, bl_sum, g, bb)


def kernel(x_occ, x_chord, x_sec, x_note, x_song, next_src, next_dst,
           instance_of_src, instance_of_dst, inst_rev_src, inst_rev_dst,
           in_section_src, in_section_dst, sec_rev_src, sec_rev_dst,
           next_section_src, next_section_dst, chord_contains_src,
           chord_contains_dst, note_in_chord_src, note_in_chord_dst,
           belongs_to_src, belongs_to_dst, song_rev_src, song_rev_dst,
           params):
    xs = {"occ": x_occ, "chord": x_chord, "sec": x_sec, "note": x_note,
          "song": x_song}
    edges = {
        "next": (next_src, next_dst),
        "instance_of": (instance_of_src, instance_of_dst),
        "inst_rev": (inst_rev_src, inst_rev_dst),
        "in_section": (in_section_src, in_section_dst),
        "sec_rev": (sec_rev_src, sec_rev_dst),
        "next_section": (next_section_src, next_section_dst),
        "chord_contains": (chord_contains_src, chord_contains_dst),
        "note_in_chord": (note_in_chord_src, note_in_chord_dst),
        "belongs_to": (belongs_to_src, belongs_to_dst),
        "song_rev": (song_rev_src, song_rev_dst),
    }
    rel_src_t = {r: s for r, s, _ in REL_LIST}
    rel_dst_t = {r: d for r, _, d in REL_LIST}
    plan = {r: _edge_plan(REL_E[r]) for r in REL_E}

    h = {}
    for t in NODE_N:
        n, f, npad = NODE_N[t], NODE_FEAT[t], NODE_PAD[t]
        xp = jnp.pad(xs[t], ((0, npad - n), (0, H - f)))
        wp = jnp.pad(params["proj"][t]["W"], ((0, H - f), (0, 0)))
        bp = params["proj"][t]["b"][None, :]
        blk = 512 if t == "occ" else npad
        h[t] = _tc_linear(xp, wp, bp, blk)

    src_idx = {}
    dst_idx = {}
    for r in REL_E:
        src, dst = edges[r]
        e = REL_E[r]
        _, _, epad = plan[r]
        d_t = rel_dst_t[r]
        src_idx[r] = jnp.pad(src, (0, epad - e))
        dstp = jnp.pad(dst, (0, epad - e), constant_values=-1)
        if d_t == "occ":
            chunks = []
            for ci in range(OCC_CHUNKS):
                lo = ci * OCC_ROWS
                loc = dstp - lo
                keep = (loc >= 0) & (loc < OCC_ROWS)
                chunks.append(jnp.where(keep, loc, OCC_ROWS).astype(jnp.int32))
            dst_idx[r] = chunks
        else:
            dst_idx[r] = [jnp.where(dstp < 0, NODE_N[d_t],
                                    dstp).astype(jnp.int32)]

    zeros = {}
    def z(npad):
        if npad not in zeros:
            zeros[npad] = jnp.zeros((npad, H), jnp.float32)
        return zeros[npad]

    def acc_pad(d_t):
        return OCC_CPAD if d_t == "occ" else NODE_PAD[d_t]

    ones_tab = jnp.ones((16, H), jnp.float32)
    zidx = {}
    for r in REL_E:
        epad = plan[r][2]
        if epad not in zidx:
            zidx[epad] = jnp.zeros((epad,), jnp.int32)
    cnt = {}
    for r in REL_E:
        b, nblk, epad = plan[r]
        npad_d = acc_pad(rel_dst_t[r])
        cnt[r] = [_sc_segsum(ones_tab, zidx[epad], dch, z(npad_d),
                             npad_d, b, nblk) for dch in dst_idx[r]]

    layer_w = []
    for li in range(NUM_LAYERS):
        lw = {}
        for t, rels in DST_RELS.items():
            lp = params["layers"][li]
            lw[t] = {
                "wl": jnp.stack([lp[r]["Wl"] for r in rels]),
                "wr": sum(lp[r]["Wr"] for r in rels),
                "bs": sum(lp[r]["bl"] for r in rels)[None, :],
                "g": params["norms"][li]["g"][None, :],
                "b": params["norms"][li]["b"][None, :],
            }
        layer_w.append(lw)

    for li in range(NUM_LAYERS):
        aggs = {}
        for r in REL_E:
            s_t = rel_src_t[r]
            b, nblk, _ = plan[r]
            npad_d = acc_pad(rel_dst_t[r])
            aggs[r] = [_sc_segsum(h[s_t], src_idx[r], dch, z(npad_d),
                                  npad_d, b, nblk) for dch in dst_idx[r]]
        new_h = {}
        for t, rels in DST_RELS.items():
            lw = layer_w[li][t]
            if t == "occ":
                parts = []
                for ci in range(OCC_CHUNKS):
                    lo = ci * OCC_ROWS
                    nrows = _occ_chunk_rows(ci)
                    parts.append(_tc_combine(
                        [aggs[r][ci] for r in rels],
                        [cnt[r][ci] for r in rels],
                        lax.slice_in_dim(h[t], lo, lo + nrows),
                        lw["wl"], lw["wr"], lw["bs"], lw["g"], lw["b"],
                        nrows, 256))
                new_h[t] = jnp.concatenate(parts, axis=0)
            else:
                new_h[t] = _tc_combine(
                    [aggs[r][0] for r in rels], [cnt[r][0] for r in rels],
                    h[t], lw["wl"], lw["wr"], lw["bs"], lw["g"], lw["b"],
                    NODE_PAD[t], NODE_PAD[t])
        h = new_h

    wc = jnp.pad(params["clf"]["W"], ((0, 0), (0, H - NUM_CLASSES)))
    bc = jnp.pad(params["clf"]["b"], (0, H - NUM_CLASSES))[None, :]
    logits = _tc_linear(h["occ"], wc, bc, 512)
    return logits[:NODE_N["occ"], :NUM_CLASSES]
